# baseline (speedup 1.0000x reference)
import functools

import jax
import jax.numpy as jnp
from jax import lax
from jax.experimental import pallas as pl
from jax.experimental.pallas import tpu as pltpu
from jax.experimental.pallas import tpu_sc as plsc

N = 10000
D = 128
E = 320000
NC = 2
NS = 16
NW = NC * NS
C = 128
K = (E + NW * C - 1) // (NW * C)
EP = NW * C * K
NP = 10240
RPT = NP // NS
RED = NP // NS


def _edge_pass(d):
  mesh = plsc.VectorSubcoreMesh(core_axis_name="c", subcore_axis_name="s")

  @functools.partial(
      pl.kernel,
      mesh=mesh,
      out_type=jax.ShapeDtypeStruct((NC * NP, d), jnp.float32),
      scratch_types=(
          [pltpu.VMEM((K * C,), jnp.int32),
           pltpu.VMEM((K, C), jnp.int32),
           pltpu.VMEM((C, d), jnp.float32),
           pltpu.VMEM_SHARED((NP, d), jnp.float32),
           pltpu.SemaphoreType.DMA]
      ),
  )
  def kern(h_hbm, src_hbm, dst_hbm, zeros_hbm, out_hbm,
           src_v, dst_v, buf, acc, sem):
    core = lax.axis_index("c")
    sub = lax.axis_index("s")
    w = core * NS + sub
    r0 = sub * RPT

    pltpu.sync_copy(zeros_hbm, acc.at[pl.ds(r0, RPT)])
    pltpu.sync_copy(src_hbm.at[w], src_v)
    pltpu.sync_copy(dst_hbm.at[w], dst_v)
    plsc.subcore_barrier()

    def body(j, carry):
      off = pl.multiple_of(j * C, 8)
      pltpu.async_copy(h_hbm.at[src_v.at[pl.ds(off, C)]], buf, sem).wait()
      pltpu.sync_copy(buf, acc.at[dst_v.at[j]], add=True)
      return carry

    lax.fori_loop(0, K, body, 0)
    plsc.subcore_barrier()
    pltpu.sync_copy(acc.at[pl.ds(r0, RPT)],
                    out_hbm.at[pl.ds(core * NP + r0, RPT)])

  return kern


def _deg_pass():
  mesh = plsc.VectorSubcoreMesh(core_axis_name="c", subcore_axis_name="s")

  @functools.partial(
      pl.kernel,
      mesh=mesh,
      out_type=jax.ShapeDtypeStruct((NC, NP, D), jnp.float32),
      scratch_types=[
          pltpu.VMEM((K, C), jnp.int32),
          pltpu.VMEM((C, D), jnp.float32),
          pltpu.VMEM_SHARED((NP, D), jnp.float32),
          pltpu.SemaphoreType.DMA,
      ],
  )
  def kern(dst_hbm, ones_hbm, zeros_hbm, out_hbm, dst_v, ones_v, acc, sem):
    core = lax.axis_index("c")
    sub = lax.axis_index("s")
    w = core * NS + sub
    r0 = sub * RPT
    pltpu.sync_copy(zeros_hbm, acc.at[pl.ds(r0, RPT)])
    pltpu.sync_copy(ones_hbm, ones_v)
    pltpu.sync_copy(dst_hbm.at[w], dst_v)
    plsc.subcore_barrier()

    def body(j, carry):
      pltpu.sync_copy(ones_v, acc.at[dst_v.at[j]], add=True)
      return carry

    lax.fori_loop(0, K, body, 0)
    plsc.subcore_barrier()
    pltpu.sync_copy(acc.at[pl.ds(r0, RPT)],
                    out_hbm.at[core, pl.ds(r0, RPT)])

  return kern


def _row_mask():
  rows = lax.broadcasted_iota(jnp.int32, (NP, 1), 0)
  return rows < N


def _bn_relu(s, g, be, mask):
  sm = jnp.where(mask, s, 0.0)
  mean = jnp.sum(sm, axis=0, keepdims=True) / N
  dlt = jnp.where(mask, s - mean, 0.0)
  var = jnp.sum(dlt * dlt, axis=0, keepdims=True) / N
  z = g * dlt * lax.rsqrt(var + 1e-5) + be
  z = jnp.maximum(z, 0.0)
  return jnp.where(mask, z, 0.0)


def _tc1_body(x_ref, w1_ref, deg0_ref, deg1_ref, h1p_ref, dinv_ref):
  deg = deg0_ref[...] + deg1_ref[...] + 1.0
  dinv = lax.rsqrt(jnp.maximum(deg, 1.0))
  u = jnp.dot(x_ref[...], w1_ref[...], preferred_element_type=jnp.float32)
  h1p_ref[...] = u * dinv
  dinv_ref[...] = jnp.broadcast_to(dinv, (NP, D))


def _tc2_body(p_ref, h1p_ref, dinv_ref, w2_ref, b1_ref, g1_ref, be1_ref,
              h2p_ref):
  mask = _row_mask()
  agg = p_ref[0:NP, :] + p_ref[NP:2 * NP, :] + h1p_ref[...]
  s1 = dinv_ref[...] * agg + b1_ref[...]
  z1 = _bn_relu(s1, g1_ref[...], be1_ref[...], mask)
  h2p_ref[...] = dinv_ref[...] * jnp.dot(
      z1, w2_ref[...], preferred_element_type=jnp.float32)


def _tc3_body(p_ref, h2p_ref, dinv_ref, b2_ref, g2_ref, be2_ref,
              z2_ref, pp_ref, qq_ref):
  mask = _row_mask()
  agg = p_ref[0:NP, :] + p_ref[NP:2 * NP, :] + h2p_ref[...]
  s2 = dinv_ref[...] * agg + b2_ref[...]
  z2 = _bn_relu(s2, g2_ref[...], be2_ref[...], mask)
  m = z2 + 1e-7
  c = jnp.max(jnp.where(mask, m, 0.0), axis=0, keepdims=True)
  pp = jnp.exp(m - c)
  z2_ref[...] = z2
  pp_ref[...] = pp
  qq_ref[...] = pp * m


def _tc4_body(pd_ref, pn_ref, z2_ref, wm1_ref, bm1_ref, wm2_ref, bm2_ref,
              out_ref):
  den = pd_ref[0:NP, :] + pd_ref[NP:2 * NP, :]
  num = pn_ref[0:NP, :] + pn_ref[NP:2 * NP, :]
  aggr = num / (den + 1e-16)
  h = z2_ref[...] + aggr
  t = jnp.maximum(
      jnp.dot(h, wm1_ref[...], preferred_element_type=jnp.float32)
      + bm1_ref[...], 0.0)
  out_ref[...] = jnp.dot(
      t, wm2_ref[...], preferred_element_type=jnp.float32) + bm2_ref[...]


def _tc(body, n_out, out_shapes):
  return pl.pallas_call(
      body,
      out_shape=[jax.ShapeDtypeStruct(s, jnp.float32) for s in out_shapes],
  )


def kernel(x, edge_index, W1, b1, W2, b2, g1, be1, g2, be2, Wm1, bm1,
           Wm2, bm2):
  f32 = jnp.float32
  x_p = jnp.zeros((NP, D), f32).at[:N].set(x)
  pad = EP - E
  src = jnp.concatenate(
      [edge_index[0], jnp.full((pad,), N, jnp.int32)]).reshape(NW, K * C)
  dst = jnp.concatenate(
      [edge_index[1], jnp.full((pad,), N, jnp.int32)]).reshape(NW, K, C)

  zerosD = jnp.zeros((RPT, D), f32)

  epD = _edge_pass(D)

  degp = _deg_pass()(dst, jnp.ones((C, D), f32), zerosD)

  tc1 = _tc(_tc1_body, 2, [(NP, D), (NP, D)])
  h1p, dinv = tc1(x_p, W1, degp[0, :, 0:1], degp[1, :, 0:1])

  p1 = epD(h1p, src, dst, zerosD)

  b1r, g1r, be1r = b1.reshape(1, D), g1.reshape(1, D), be1.reshape(1, D)
  b2r, g2r, be2r = b2.reshape(1, D), g2.reshape(1, D), be2.reshape(1, D)
  tc2 = _tc(_tc2_body, 1, [(NP, D)])
  (h2p,) = tc2(p1, h1p, dinv, W2, b1r, g1r, be1r)

  p2 = epD(h2p, src, dst, zerosD)

  tc3 = _tc(_tc3_body, 3, [(NP, D), (NP, D), (NP, D)])
  z2, pp, qq = tc3(p2, h2p, dinv, b2r, g2r, be2r)

  pd = epD(pp, src, dst, zerosD)
  pn = epD(qq, src, dst, zerosD)

  tc4 = _tc(_tc4_body, 1, [(NP, D)])
  (outp,) = tc4(pd, pn, z2, Wm1, bm1.reshape(1, 2 * D), Wm2,
                bm2.reshape(1, D))
  return outp[:N]


# speedup vs baseline: 1.7865x; 1.7865x over previous
import functools

import jax
import jax.numpy as jnp
from jax import lax
from jax.experimental import pallas as pl
from jax.experimental.pallas import tpu as pltpu
from jax.experimental.pallas import tpu_sc as plsc

N = 10000
D = 128
E = 320000
NC = 2
NS = 16
NW = NC * NS
C = 128
K = (E + NW * C - 1) // (NW * C)
EP = NW * C * K
NP = 10240
RPT = NP // NS
RED = NP // NS


def _edge_pass(d):
  mesh = plsc.VectorSubcoreMesh(core_axis_name="c", subcore_axis_name="s")

  @functools.partial(
      pl.kernel,
      mesh=mesh,
      out_type=jax.ShapeDtypeStruct((NC * NP, d), jnp.float32),
      scratch_types=(
          [pltpu.VMEM((K * C,), jnp.int32),
           pltpu.VMEM((K, C), jnp.int32),
           pltpu.VMEM((C, d), jnp.float32),
           pltpu.VMEM_SHARED((NP, d), jnp.float32),
           pltpu.SemaphoreType.DMA]
      ),
  )
  def kern(h_hbm, src_hbm, dst_hbm, zeros_hbm, out_hbm,
           src_v, dst_v, buf, acc, sem):
    core = lax.axis_index("c")
    sub = lax.axis_index("s")
    w = core * NS + sub
    r0 = sub * RPT

    pltpu.sync_copy(zeros_hbm, acc.at[pl.ds(r0, RPT)])
    pltpu.sync_copy(src_hbm.at[w], src_v)
    pltpu.sync_copy(dst_hbm.at[w], dst_v)
    plsc.subcore_barrier()

    def body(j, carry):
      off = pl.multiple_of(j * C, 8)
      pltpu.async_copy(h_hbm.at[src_v.at[pl.ds(off, C)]], buf, sem).wait()
      pltpu.sync_copy(buf, acc.at[dst_v.at[j]], add=True)
      return carry

    lax.fori_loop(0, K, body, 0)
    plsc.subcore_barrier()
    pltpu.sync_copy(acc.at[pl.ds(r0, RPT)],
                    out_hbm.at[pl.ds(core * NP + r0, RPT)])

  return kern


def _deg_pass():
  mesh = plsc.VectorSubcoreMesh(core_axis_name="c", subcore_axis_name="s")

  @functools.partial(
      pl.kernel,
      mesh=mesh,
      out_type=jax.ShapeDtypeStruct((NC, NP, D), jnp.float32),
      scratch_types=[
          pltpu.VMEM((K, C), jnp.int32),
          pltpu.VMEM((C, D), jnp.float32),
          pltpu.VMEM_SHARED((NP, D), jnp.float32),
          pltpu.SemaphoreType.DMA,
      ],
  )
  def kern(dst_hbm, ones_hbm, zeros_hbm, out_hbm, dst_v, ones_v, acc, sem):
    core = lax.axis_index("c")
    sub = lax.axis_index("s")
    w = core * NS + sub
    r0 = sub * RPT
    pltpu.sync_copy(zeros_hbm, acc.at[pl.ds(r0, RPT)])
    pltpu.sync_copy(ones_hbm, ones_v)
    pltpu.sync_copy(dst_hbm.at[w], dst_v)
    plsc.subcore_barrier()

    def body(j, carry):
      pltpu.sync_copy(ones_v, acc.at[dst_v.at[j]], add=True)
      return carry

    lax.fori_loop(0, K, body, 0)
    plsc.subcore_barrier()
    pltpu.sync_copy(acc.at[pl.ds(r0, RPT)],
                    out_hbm.at[core, pl.ds(r0, RPT)])

  return kern


def _row_mask():
  rows = lax.broadcasted_iota(jnp.int32, (NP, 1), 0)
  return rows < N


def _bn_relu(s, g, be, mask):
  sm = jnp.where(mask, s, 0.0)
  mean = jnp.sum(sm, axis=0, keepdims=True) / N
  dlt = jnp.where(mask, s - mean, 0.0)
  var = jnp.sum(dlt * dlt, axis=0, keepdims=True) / N
  z = g * dlt * lax.rsqrt(var + 1e-5) + be
  z = jnp.maximum(z, 0.0)
  return jnp.where(mask, z, 0.0)


def _tc1_body(x_ref, w1_ref, deg0_ref, deg1_ref, h1p_ref, dinv_ref):
  deg = deg0_ref[...] + deg1_ref[...] + 1.0
  dinv = lax.rsqrt(jnp.maximum(deg, 1.0))
  u = jnp.dot(x_ref[...], w1_ref[...], preferred_element_type=jnp.float32)
  h1p_ref[...] = u * dinv
  dinv_ref[...] = jnp.broadcast_to(dinv, (NP, D))


def _tc2_body(p_ref, h1p_ref, dinv_ref, w2_ref, b1_ref, g1_ref, be1_ref,
              h2p_ref):
  mask = _row_mask()
  agg = p_ref[0:NP, :] + p_ref[NP:2 * NP, :] + h1p_ref[...]
  s1 = dinv_ref[...] * agg + b1_ref[...]
  z1 = _bn_relu(s1, g1_ref[...], be1_ref[...], mask)
  h2p_ref[...] = dinv_ref[...] * jnp.dot(
      z1, w2_ref[...], preferred_element_type=jnp.float32)


def _tc3_body(p_ref, h2p_ref, dinv_ref, b2_ref, g2_ref, be2_ref,
              z2_ref, pp_ref, qq_ref):
  mask = _row_mask()
  agg = p_ref[0:NP, :] + p_ref[NP:2 * NP, :] + h2p_ref[...]
  s2 = dinv_ref[...] * agg + b2_ref[...]
  z2 = _bn_relu(s2, g2_ref[...], be2_ref[...], mask)
  m = z2 + 1e-7
  c = jnp.max(jnp.where(mask, m, 0.0), axis=0, keepdims=True)
  pp = jnp.exp(m - c)
  z2_ref[...] = z2
  pp_ref[...] = pp
  qq_ref[...] = pp * m


def _tc4_body(pd_ref, pn_ref, z2_ref, wm1_ref, bm1_ref, wm2_ref, bm2_ref,
              out_ref):
  den = pd_ref[0:NP, :] + pd_ref[NP:2 * NP, :]
  num = pn_ref[0:NP, :] + pn_ref[NP:2 * NP, :]
  aggr = num / (den + 1e-16)
  h = z2_ref[...] + aggr
  t = jnp.maximum(
      jnp.dot(h, wm1_ref[...], preferred_element_type=jnp.float32)
      + bm1_ref[...], 0.0)
  out_ref[...] = jnp.dot(
      t, wm2_ref[...], preferred_element_type=jnp.float32) + bm2_ref[...]


def _tc(body, n_out, out_shapes):
  return pl.pallas_call(
      body,
      out_shape=[jax.ShapeDtypeStruct(s, jnp.float32) for s in out_shapes],
  )


def kernel(x, edge_index, W1, b1, W2, b2, g1, be1, g2, be2, Wm1, bm1,
           Wm2, bm2):
  f32 = jnp.float32
  x_p = jnp.zeros((NP, D), f32).at[:N].set(x)
  pad = EP - E
  pad_idx = N + jnp.arange(pad, dtype=jnp.int32) % (NP - N)
  src = jnp.concatenate([edge_index[0], pad_idx]).reshape(NW, K * C)
  dst = jnp.concatenate([edge_index[1], pad_idx]).reshape(NW, K, C)

  zerosD = jnp.zeros((RPT, D), f32)

  epD = _edge_pass(D)

  degp = _deg_pass()(dst, jnp.ones((C, D), f32), zerosD)

  tc1 = _tc(_tc1_body, 2, [(NP, D), (NP, D)])
  h1p, dinv = tc1(x_p, W1, degp[0, :, 0:1], degp[1, :, 0:1])

  p1 = epD(h1p, src, dst, zerosD)

  b1r, g1r, be1r = b1.reshape(1, D), g1.reshape(1, D), be1.reshape(1, D)
  b2r, g2r, be2r = b2.reshape(1, D), g2.reshape(1, D), be2.reshape(1, D)
  tc2 = _tc(_tc2_body, 1, [(NP, D)])
  (h2p,) = tc2(p1, h1p, dinv, W2, b1r, g1r, be1r)

  p2 = epD(h2p, src, dst, zerosD)

  tc3 = _tc(_tc3_body, 3, [(NP, D), (NP, D), (NP, D)])
  z2, pp, qq = tc3(p2, h2p, dinv, b2r, g2r, be2r)

  pd = epD(pp, src, dst, zerosD)
  pn = epD(qq, src, dst, zerosD)

  tc4 = _tc(_tc4_body, 1, [(NP, D)])
  (outp,) = tc4(pd, pn, z2, Wm1, bm1.reshape(1, 2 * D), Wm2,
                bm2.reshape(1, D))
  return outp[:N]


# speedup vs baseline: 2.2293x; 1.2478x over previous
import functools

import jax
import jax.numpy as jnp
from jax import lax
from jax.experimental import pallas as pl
from jax.experimental.pallas import tpu as pltpu
from jax.experimental.pallas import tpu_sc as plsc

N = 10000
D = 128
E = 320000
NC = 2
NS = 16
NW = NC * NS
C = 128
K = ((E + NW * C - 1) // (NW * C) + 3) // 4 * 4
KH = K // 2
EP = NW * C * K
NP = 10240
RPT = NP // NS
RED = NP // NS


def _edge_pass(d):
  mesh = plsc.VectorSubcoreMesh(core_axis_name="c", subcore_axis_name="s")

  @functools.partial(
      pl.kernel,
      mesh=mesh,
      out_type=jax.ShapeDtypeStruct((NC * NP, d), jnp.float32),
      scratch_types=(
          [pltpu.VMEM((KH * C,), jnp.int32),
           pltpu.VMEM((K, C), jnp.int32),
           pltpu.VMEM((C, d), jnp.float32),
           pltpu.VMEM((C, d), jnp.float32),
           pltpu.VMEM_SHARED((NP, d), jnp.float32)]
          + [pltpu.SemaphoreType.DMA] * 4
      ),
  )
  def kern(h_hbm, src_hbm, dst_hbm, zeros_hbm, out_hbm,
           src_v, dst_v, b0, b1, acc, gs0, gs1, ss0, ss1):
    bufs = (b0, b1)
    gsems = (gs0, gs1)
    ssems = (ss0, ss1)
    core = lax.axis_index("c")
    sub = lax.axis_index("s")
    w = core * NS + sub
    r0 = sub * RPT

    def g_start(j, u):
      off = pl.multiple_of(j * C, 8)
      pltpu.async_copy(h_hbm.at[src_v.at[pl.ds(off, C)]], bufs[u],
                       gsems[u])

    def g_wait(j, u):
      off = pl.multiple_of(j * C, 8)
      pltpu.make_async_copy(h_hbm.at[src_v.at[pl.ds(off, C)]], bufs[u],
                            gsems[u]).wait()

    def s_start(gj, u):
      pltpu.async_copy(bufs[u], acc.at[dst_v.at[gj]], ssems[u], add=True)

    def s_wait(gj, u):
      pltpu.make_async_copy(bufs[u], acc.at[dst_v.at[gj]], ssems[u]).wait()

    pltpu.sync_copy(zeros_hbm, acc.at[pl.ds(r0, RPT)])
    pltpu.sync_copy(dst_hbm.at[w], dst_v)
    plsc.subcore_barrier()

    for h in range(2):
      B = h * KH
      pltpu.sync_copy(src_hbm.at[w, pl.ds(B * C, KH * C)], src_v)
      g_start(0, 0)
      g_wait(0, 0)
      s_start(B, 0)
      g_start(1, 1)
      g_wait(1, 1)
      s_start(B + 1, 1)
      s_wait(B, 0)
      g_start(2, 0)

      def body(g, carry):
        for u in range(2):
          j = 2 * g + u
          g_wait(j, u)
          s_start(B + j, u)
          s_wait(B + j - 1, 1 - u)
          g_start(j + 1, 1 - u)
        return carry

      lax.fori_loop(1, KH // 2 - 1, body, 0)

      for u in range(2):
        j = KH - 2 + u
        g_wait(j, u)
        s_start(B + j, u)
        s_wait(B + j - 1, 1 - u)
        if j + 1 < KH:
          g_start(j + 1, 1 - u)
      s_wait(B + KH - 1, 1)

    plsc.subcore_barrier()
    pltpu.sync_copy(acc.at[pl.ds(r0, RPT)],
                    out_hbm.at[pl.ds(core * NP + r0, RPT)])

  return kern


def _deg_pass():
  mesh = plsc.VectorSubcoreMesh(core_axis_name="c", subcore_axis_name="s")

  @functools.partial(
      pl.kernel,
      mesh=mesh,
      out_type=jax.ShapeDtypeStruct((NC, NP, D), jnp.float32),
      scratch_types=[
          pltpu.VMEM((K, C), jnp.int32),
          pltpu.VMEM((C, D), jnp.float32),
          pltpu.VMEM_SHARED((NP, D), jnp.float32),
          pltpu.SemaphoreType.DMA,
      ],
  )
  def kern(dst_hbm, ones_hbm, zeros_hbm, out_hbm, dst_v, ones_v, acc, sem):
    core = lax.axis_index("c")
    sub = lax.axis_index("s")
    w = core * NS + sub
    r0 = sub * RPT
    pltpu.sync_copy(zeros_hbm, acc.at[pl.ds(r0, RPT)])
    pltpu.sync_copy(ones_hbm, ones_v)
    pltpu.sync_copy(dst_hbm.at[w], dst_v)
    plsc.subcore_barrier()

    def body(j, carry):
      pltpu.sync_copy(ones_v, acc.at[dst_v.at[j]], add=True)
      return carry

    lax.fori_loop(0, K, body, 0)
    plsc.subcore_barrier()
    pltpu.sync_copy(acc.at[pl.ds(r0, RPT)],
                    out_hbm.at[core, pl.ds(r0, RPT)])

  return kern


def _row_mask():
  rows = lax.broadcasted_iota(jnp.int32, (NP, 1), 0)
  return rows < N


def _bn_relu(s, g, be, mask):
  sm = jnp.where(mask, s, 0.0)
  mean = jnp.sum(sm, axis=0, keepdims=True) / N
  dlt = jnp.where(mask, s - mean, 0.0)
  var = jnp.sum(dlt * dlt, axis=0, keepdims=True) / N
  z = g * dlt * lax.rsqrt(var + 1e-5) + be
  z = jnp.maximum(z, 0.0)
  return jnp.where(mask, z, 0.0)


def _tc1_body(x_ref, w1_ref, deg0_ref, deg1_ref, h1p_ref, dinv_ref):
  deg = deg0_ref[...] + deg1_ref[...] + 1.0
  dinv = lax.rsqrt(jnp.maximum(deg, 1.0))
  u = jnp.dot(x_ref[...], w1_ref[...], preferred_element_type=jnp.float32)
  h1p_ref[...] = u * dinv
  dinv_ref[...] = jnp.broadcast_to(dinv, (NP, D))


def _tc2_body(p_ref, h1p_ref, dinv_ref, w2_ref, b1_ref, g1_ref, be1_ref,
              h2p_ref):
  mask = _row_mask()
  agg = p_ref[0:NP, :] + p_ref[NP:2 * NP, :] + h1p_ref[...]
  s1 = dinv_ref[...] * agg + b1_ref[...]
  z1 = _bn_relu(s1, g1_ref[...], be1_ref[...], mask)
  h2p_ref[...] = dinv_ref[...] * jnp.dot(
      z1, w2_ref[...], preferred_element_type=jnp.float32)


def _tc3_body(p_ref, h2p_ref, dinv_ref, b2_ref, g2_ref, be2_ref,
              z2_ref, pp_ref, qq_ref):
  mask = _row_mask()
  agg = p_ref[0:NP, :] + p_ref[NP:2 * NP, :] + h2p_ref[...]
  s2 = dinv_ref[...] * agg + b2_ref[...]
  z2 = _bn_relu(s2, g2_ref[...], be2_ref[...], mask)
  m = z2 + 1e-7
  c = jnp.max(jnp.where(mask, m, 0.0), axis=0, keepdims=True)
  pp = jnp.exp(m - c)
  z2_ref[...] = z2
  pp_ref[...] = pp
  qq_ref[...] = pp * m


def _tc4_body(pd_ref, pn_ref, z2_ref, wm1_ref, bm1_ref, wm2_ref, bm2_ref,
              out_ref):
  den = pd_ref[0:NP, :] + pd_ref[NP:2 * NP, :]
  num = pn_ref[0:NP, :] + pn_ref[NP:2 * NP, :]
  aggr = num / (den + 1e-16)
  h = z2_ref[...] + aggr
  t = jnp.maximum(
      jnp.dot(h, wm1_ref[...], preferred_element_type=jnp.float32)
      + bm1_ref[...], 0.0)
  out_ref[...] = jnp.dot(
      t, wm2_ref[...], preferred_element_type=jnp.float32) + bm2_ref[...]


def _tc(body, n_out, out_shapes):
  return pl.pallas_call(
      body,
      out_shape=[jax.ShapeDtypeStruct(s, jnp.float32) for s in out_shapes],
  )


def kernel(x, edge_index, W1, b1, W2, b2, g1, be1, g2, be2, Wm1, bm1,
           Wm2, bm2):
  f32 = jnp.float32
  x_p = jnp.zeros((NP, D), f32).at[:N].set(x)
  pad = EP - E
  pad_idx = N + jnp.arange(pad, dtype=jnp.int32) % (NP - N)
  src = jnp.concatenate([edge_index[0], pad_idx]).reshape(NW, K * C)
  dst = jnp.concatenate([edge_index[1], pad_idx]).reshape(NW, K, C)

  zerosD = jnp.zeros((RPT, D), f32)

  epD = _edge_pass(D)

  degp = _deg_pass()(dst, jnp.ones((C, D), f32), zerosD)

  tc1 = _tc(_tc1_body, 2, [(NP, D), (NP, D)])
  h1p, dinv = tc1(x_p, W1, degp[0, :, 0:1], degp[1, :, 0:1])

  p1 = epD(h1p, src, dst, zerosD)

  b1r, g1r, be1r = b1.reshape(1, D), g1.reshape(1, D), be1.reshape(1, D)
  b2r, g2r, be2r = b2.reshape(1, D), g2.reshape(1, D), be2.reshape(1, D)
  tc2 = _tc(_tc2_body, 1, [(NP, D)])
  (h2p,) = tc2(p1, h1p, dinv, W2, b1r, g1r, be1r)

  p2 = epD(h2p, src, dst, zerosD)

  tc3 = _tc(_tc3_body, 3, [(NP, D), (NP, D), (NP, D)])
  z2, pp, qq = tc3(p2, h2p, dinv, b2r, g2r, be2r)

  pd = epD(pp, src, dst, zerosD)
  pn = epD(qq, src, dst, zerosD)

  tc4 = _tc(_tc4_body, 1, [(NP, D)])
  (outp,) = tc4(pd, pn, z2, Wm1, bm1.reshape(1, 2 * D), Wm2,
                bm2.reshape(1, D))
  return outp[:N]


# speedup vs baseline: 2.2313x; 1.0009x over previous
import functools

import jax
import jax.numpy as jnp
from jax import lax
from jax.experimental import pallas as pl
from jax.experimental.pallas import tpu as pltpu
from jax.experimental.pallas import tpu_sc as plsc

N = 10000
D = 128
E = 320000
NC = 2
NS = 16
NW = NC * NS
C = 128
K = ((E + NW * C - 1) // (NW * C) + 3) // 4 * 4
KH = K // 2
EP = NW * C * K
NP = 10240
RPT = NP // NS
RED = NP // NS


def _edge_pass(d):
  mesh = plsc.VectorSubcoreMesh(core_axis_name="c", subcore_axis_name="s")

  @functools.partial(
      pl.kernel,
      mesh=mesh,
      out_type=jax.ShapeDtypeStruct((NC * NP, d), jnp.float32),
      scratch_types=(
          [pltpu.VMEM((KH * C,), jnp.int32),
           pltpu.VMEM((K, C), jnp.int32),
           pltpu.VMEM((C, d), jnp.float32),
           pltpu.VMEM((C, d), jnp.float32),
           pltpu.VMEM_SHARED((NP, d), jnp.float32)]
          + [pltpu.SemaphoreType.DMA] * 4
      ),
  )
  def kern(h_hbm, src_hbm, dst_hbm, zeros_hbm, out_hbm,
           src_v, dst_v, b0, b1, acc, gs0, gs1, ss0, ss1):
    bufs = (b0, b1)
    gsems = (gs0, gs1)
    ssems = (ss0, ss1)
    core = lax.axis_index("c")
    sub = lax.axis_index("s")
    w = core * NS + sub
    r0 = sub * RPT

    def g_start(j, u):
      off = pl.multiple_of(j * C, 8)
      pltpu.async_copy(h_hbm.at[src_v.at[pl.ds(off, C)]], bufs[u],
                       gsems[u])

    def g_wait(j, u):
      off = pl.multiple_of(j * C, 8)
      pltpu.make_async_copy(h_hbm.at[src_v.at[pl.ds(off, C)]], bufs[u],
                            gsems[u]).wait()

    def s_start(gj, u):
      pltpu.async_copy(bufs[u], acc.at[dst_v.at[gj]], ssems[u], add=True)

    def s_wait(gj, u):
      pltpu.make_async_copy(bufs[u], acc.at[dst_v.at[gj]], ssems[u]).wait()

    pltpu.sync_copy(zeros_hbm, acc.at[pl.ds(r0, RPT)])
    pltpu.sync_copy(dst_hbm.at[w], dst_v)
    plsc.subcore_barrier()

    for h in range(2):
      B = h * KH
      pltpu.sync_copy(src_hbm.at[w, pl.ds(B * C, KH * C)], src_v)
      g_start(0, 0)
      g_wait(0, 0)
      s_start(B, 0)
      g_start(1, 1)
      g_wait(1, 1)
      s_start(B + 1, 1)
      s_wait(B, 0)
      g_start(2, 0)

      def body(g, carry):
        for u in range(2):
          j = 2 * g + u
          g_wait(j, u)
          s_start(B + j, u)
          s_wait(B + j - 1, 1 - u)
          g_start(j + 1, 1 - u)
        return carry

      lax.fori_loop(1, KH // 2 - 1, body, 0)

      for u in range(2):
        j = KH - 2 + u
        g_wait(j, u)
        s_start(B + j, u)
        s_wait(B + j - 1, 1 - u)
        if j + 1 < KH:
          g_start(j + 1, 1 - u)
      s_wait(B + KH - 1, 1)

    plsc.subcore_barrier()
    pltpu.sync_copy(acc.at[pl.ds(r0, RPT)],
                    out_hbm.at[pl.ds(core * NP + r0, RPT)])

  return kern


def _deg_pass():
  mesh = plsc.VectorSubcoreMesh(core_axis_name="c", subcore_axis_name="s")

  @functools.partial(
      pl.kernel,
      mesh=mesh,
      out_type=jax.ShapeDtypeStruct((NC, NP, D), jnp.float32),
      scratch_types=[
          pltpu.VMEM((K, C), jnp.int32),
          pltpu.VMEM((C, D), jnp.float32),
          pltpu.VMEM_SHARED((NP, D), jnp.float32),
          pltpu.SemaphoreType.DMA,
      ],
  )
  def kern(dst_hbm, ones_hbm, zeros_hbm, out_hbm, dst_v, ones_v, acc, sem):
    core = lax.axis_index("c")
    sub = lax.axis_index("s")
    w = core * NS + sub
    r0 = sub * RPT
    pltpu.sync_copy(zeros_hbm, acc.at[pl.ds(r0, RPT)])
    pltpu.sync_copy(ones_hbm, ones_v)
    pltpu.sync_copy(dst_hbm.at[w], dst_v)
    plsc.subcore_barrier()

    QD = 8

    def body(j, carry):
      pltpu.async_copy(ones_v, acc.at[dst_v.at[j]], sem, add=True)

      @pl.when(j >= QD)
      def _():
        pltpu.make_async_copy(ones_v, acc.at[dst_v.at[0]], sem).wait()
      return carry

    lax.fori_loop(0, K, body, 0)
    for _ in range(QD):
      pltpu.make_async_copy(ones_v, acc.at[dst_v.at[0]], sem).wait()
    plsc.subcore_barrier()
    pltpu.sync_copy(acc.at[pl.ds(r0, RPT)],
                    out_hbm.at[core, pl.ds(r0, RPT)])

  return kern


def _row_mask():
  rows = lax.broadcasted_iota(jnp.int32, (NP, 1), 0)
  return rows < N


def _bn_relu(s, g, be, mask):
  sm = jnp.where(mask, s, 0.0)
  mean = jnp.sum(sm, axis=0, keepdims=True) / N
  dlt = jnp.where(mask, s - mean, 0.0)
  var = jnp.sum(dlt * dlt, axis=0, keepdims=True) / N
  z = g * dlt * lax.rsqrt(var + 1e-5) + be
  z = jnp.maximum(z, 0.0)
  return jnp.where(mask, z, 0.0)


def _tc1_body(x_ref, w1_ref, deg0_ref, deg1_ref, h1p_ref, dinv_ref):
  deg = deg0_ref[...] + deg1_ref[...] + 1.0
  dinv = lax.rsqrt(jnp.maximum(deg, 1.0))
  u = jnp.dot(x_ref[...], w1_ref[...], preferred_element_type=jnp.float32)
  h1p_ref[...] = u * dinv
  dinv_ref[...] = jnp.broadcast_to(dinv, (NP, D))


def _tc2_body(p_ref, h1p_ref, dinv_ref, w2_ref, b1_ref, g1_ref, be1_ref,
              h2p_ref):
  mask = _row_mask()
  agg = p_ref[0:NP, :] + p_ref[NP:2 * NP, :] + h1p_ref[...]
  s1 = dinv_ref[...] * agg + b1_ref[...]
  z1 = _bn_relu(s1, g1_ref[...], be1_ref[...], mask)
  h2p_ref[...] = dinv_ref[...] * jnp.dot(
      z1, w2_ref[...], preferred_element_type=jnp.float32)


def _tc3_body(p_ref, h2p_ref, dinv_ref, b2_ref, g2_ref, be2_ref,
              z2_ref, pp_ref, qq_ref):
  mask = _row_mask()
  agg = p_ref[0:NP, :] + p_ref[NP:2 * NP, :] + h2p_ref[...]
  s2 = dinv_ref[...] * agg + b2_ref[...]
  z2 = _bn_relu(s2, g2_ref[...], be2_ref[...], mask)
  m = z2 + 1e-7
  c = jnp.max(jnp.where(mask, m, 0.0), axis=0, keepdims=True)
  pp = jnp.exp(m - c)
  z2_ref[...] = z2
  pp_ref[...] = pp
  qq_ref[...] = pp * m


def _tc4_body(pd_ref, pn_ref, z2_ref, wm1_ref, bm1_ref, wm2_ref, bm2_ref,
              out_ref):
  den = pd_ref[0:NP, :] + pd_ref[NP:2 * NP, :]
  num = pn_ref[0:NP, :] + pn_ref[NP:2 * NP, :]
  aggr = num / (den + 1e-16)
  h = z2_ref[...] + aggr
  t = jnp.maximum(
      jnp.dot(h, wm1_ref[...], preferred_element_type=jnp.float32)
      + bm1_ref[...], 0.0)
  out_ref[...] = jnp.dot(
      t, wm2_ref[...], preferred_element_type=jnp.float32) + bm2_ref[...]


def _tc(body, n_out, out_shapes):
  return pl.pallas_call(
      body,
      out_shape=[jax.ShapeDtypeStruct(s, jnp.float32) for s in out_shapes],
  )


def kernel(x, edge_index, W1, b1, W2, b2, g1, be1, g2, be2, Wm1, bm1,
           Wm2, bm2):
  f32 = jnp.float32
  x_p = jnp.zeros((NP, D), f32).at[:N].set(x)
  pad = EP - E
  pad_idx = N + jnp.arange(pad, dtype=jnp.int32) % (NP - N)
  src = jnp.concatenate([edge_index[0], pad_idx]).reshape(NW, K * C)
  dst = jnp.concatenate([edge_index[1], pad_idx]).reshape(NW, K, C)

  zerosD = jnp.zeros((RPT, D), f32)

  epD = _edge_pass(D)

  degp = _deg_pass()(dst, jnp.ones((C, D), f32), zerosD)

  tc1 = _tc(_tc1_body, 2, [(NP, D), (NP, D)])
  h1p, dinv = tc1(x_p, W1, degp[0, :, 0:1], degp[1, :, 0:1])

  p1 = epD(h1p, src, dst, zerosD)

  b1r, g1r, be1r = b1.reshape(1, D), g1.reshape(1, D), be1.reshape(1, D)
  b2r, g2r, be2r = b2.reshape(1, D), g2.reshape(1, D), be2.reshape(1, D)
  tc2 = _tc(_tc2_body, 1, [(NP, D)])
  (h2p,) = tc2(p1, h1p, dinv, W2, b1r, g1r, be1r)

  p2 = epD(h2p, src, dst, zerosD)

  tc3 = _tc(_tc3_body, 3, [(NP, D), (NP, D), (NP, D)])
  z2, pp, qq = tc3(p2, h2p, dinv, b2r, g2r, be2r)

  pd = epD(pp, src, dst, zerosD)
  pn = epD(qq, src, dst, zerosD)

  tc4 = _tc(_tc4_body, 1, [(NP, D)])
  (outp,) = tc4(pd, pn, z2, Wm1, bm1.reshape(1, 2 * D), Wm2,
                bm2.reshape(1, D))
  return outp[:N]
